# Initial kernel scaffold; baseline (speedup 1.0000x reference)
import jax
import jax.numpy as jnp
from jax.experimental import pallas as pl


def kernel(x, edge_index, edge_values):
    raise NotImplementedError("write your pallas kernel here")


# speedup vs baseline: 4.0778x; 4.0778x over previous
import functools

import jax
import jax.numpy as jnp
from jax import lax
from jax.experimental import pallas as pl
from jax.experimental.pallas import tpu as pltpu
from jax.experimental.pallas import tpu_sc as plsc

NC = 2
NS = 16
L = 16
NW = NC * NS


def _sc_spmm(x, srcp, dstp, wp, zeros, n, d, nb):
    mesh = plsc.VectorSubcoreMesh(
        core_axis_name="c", subcore_axis_name="s", num_cores=NC, num_subcores=NS
    )

    @functools.partial(
        pl.kernel,
        out_type=jax.ShapeDtypeStruct((NC, n, d), jnp.float32),
        mesh=mesh,
        compiler_params=pltpu.CompilerParams(needs_layout_passes=False),
        scratch_types=[
            pltpu.VMEM((nb, 128), jnp.int32),
            pltpu.VMEM((nb, 128), jnp.int32),
            pltpu.VMEM((nb, 128), jnp.float32),
            pltpu.VMEM((128, d), jnp.float32),
            pltpu.VMEM_SHARED((n, d), jnp.float32),
            pltpu.SemaphoreType.DMA,
            pltpu.SemaphoreType.DMA,
        ],
    )
    def k(x_hbm, src_hbm, dst_hbm, w_hbm, z_hbm, part_hbm,
          src_v, dst_v, w_v, rows_v, acc, gsem, ssem):
        c = lax.axis_index("c")
        s = lax.axis_index("s")
        wid = c * NS + s
        pltpu.sync_copy(src_hbm.at[wid], src_v)
        pltpu.sync_copy(dst_hbm.at[wid], dst_v)
        pltpu.sync_copy(w_hbm.at[wid], w_v)
        rpt = n // NS
        pltpu.sync_copy(z_hbm.at[pl.ds(s * rpt, rpt)], acc.at[pl.ds(s * rpt, rpt)])
        plsc.subcore_barrier()

        zeros16 = jnp.zeros((L,), jnp.int32)

        def batch(j, carry):
            pltpu.async_copy(x_hbm.at[src_v.at[j]], rows_v, gsem).wait()
            jsplat = zeros16 + j

            def edge(e, carry2):
                esplat = zeros16 + e
                wv = plsc.load_gather(w_v, [jsplat, esplat])
                for k8 in range(d // L):
                    sl = pl.ds(k8 * L, L)
                    rows_v[e, sl] = rows_v[e, sl] * wv
                return carry2

            lax.fori_loop(0, 128, edge, 0)
            pltpu.async_copy(rows_v, acc.at[dst_v.at[j]], ssem, add=True).wait()
            return carry

        lax.fori_loop(0, nb, batch, 0)
        plsc.subcore_barrier()
        pltpu.sync_copy(acc.at[pl.ds(s * rpt, rpt)],
                        part_hbm.at[c, pl.ds(s * rpt, rpt)])

    return k(x, srcp, dstp, wp, zeros)


def _combine(x, part):
    n, d = x.shape
    blk = 1000

    def body(x_ref, p_ref, o_ref):
        f = p_ref[0] + p_ref[1]
        o_ref[...] = x_ref[...] + jnp.maximum(f, 0.0)

    return pl.pallas_call(
        body,
        grid=(n // blk,),
        in_specs=[
            pl.BlockSpec((blk, d), lambda i: (i, 0)),
            pl.BlockSpec((NC, blk, d), lambda i: (0, i, 0)),
        ],
        out_specs=pl.BlockSpec((blk, d), lambda i: (i, 0)),
        out_shape=jax.ShapeDtypeStruct((n, d), jnp.float32),
    )(x, part)


def kernel(x, edge_index, edge_values):
    n, d = x.shape
    e = edge_values.shape[0]
    ept = -(-e // (NW * 128)) * 128
    epad = ept * NW
    nb = ept // 128
    dst = edge_index[0]
    src = edge_index[1]
    pad = epad - e
    srcp = jnp.pad(src, (0, pad)).reshape(NW, nb, 128)
    dstp = jnp.pad(dst, (0, pad)).reshape(NW, nb, 128)
    wp = jnp.pad(edge_values, (0, pad)).reshape(NW, nb, 128)
    rpt = 8 * (-(-n // (NS * 8)))
    npad = rpt * NS
    zeros = jnp.zeros((npad, d), jnp.float32)
    part = _sc_spmm(x, srcp, dstp, wp, zeros, npad, d, nb)
    return _combine(x, part)
